# Initial kernel scaffold; baseline (speedup 1.0000x reference)
import jax
import jax.numpy as jnp
from jax.experimental import pallas as pl


def kernel(inputs):
    raise NotImplementedError("write your pallas kernel here")


# speedup vs baseline: 5.3001x; 5.3001x over previous
import functools

import jax
import jax.numpy as jnp
from jax import lax
from jax.experimental import pallas as pl
from jax.experimental.pallas import tpu as pltpu
from jax.experimental.pallas import tpu_sc as plsc

_BOUNDS = [-3.1, -2.9, -2.7, -2.5, -2.3, -2.1, -1.9, -1.7, -1.5, -1.3, -1.1,
           -0.9, -0.7, -0.5, -0.3, -0.1, 0.1, 0.3, 0.5, 0.7, 0.9, 1.1, 1.3,
           1.5, 1.7, 1.9, 2.1, 2.3, 2.5, 2.7, 2.9, 3.1]

_N = 4096 * 4096
_NW = 32
_PER_W = _N // _NW
_CH = 16384
_NCH = _PER_W // _CH
_VECS = _CH // 16


def _bucketize_chunk(vin, vout, tbl):
  def body(i, _):
    v = vin[pl.ds(i * 16, 16)]
    t = v * 5.0 + 16.0
    t = jnp.minimum(jnp.maximum(t, 0.0), 31.0)
    j = t.astype(jnp.int32)
    b = plsc.load_gather(tbl, [j])
    vout[pl.ds(i * 16, 16)] = j + (v >= b).astype(jnp.int32)
    return 0

  lax.fori_loop(0, _VECS, body, 0)


def _sc_bucketize(x_hbm, tbl_hbm, out_hbm, tbl_v, vin0, vin1, vout0, vout1,
                  isem0, isem1, osem0, osem1):
  wid = lax.axis_index("s") * 2 + lax.axis_index("c")
  base = wid * _PER_W

  pltpu.sync_copy(tbl_hbm, tbl_v)

  vins = (vin0, vin1)
  vouts = (vout0, vout1)
  isems = (isem0, isem1)
  osems = (osem0, osem1)

  def start_in(c):
    slot = c % 2
    pltpu.async_copy(x_hbm.at[pl.ds(base + c * _CH, _CH)], vins[slot],
                     isems[slot])

  start_in(0)
  for c in range(_NCH):
    slot = c % 2
    if c + 1 < _NCH:
      start_in(c + 1)
    pltpu.make_async_copy(x_hbm.at[pl.ds(base + c * _CH, _CH)], vins[slot],
                          isems[slot]).wait()
    if c >= 2:
      pltpu.make_async_copy(vouts[slot],
                            out_hbm.at[pl.ds(base + (c - 2) * _CH, _CH)],
                            osems[slot]).wait()
    _bucketize_chunk(vins[slot], vouts[slot], tbl_v)
    pltpu.async_copy(vouts[slot], out_hbm.at[pl.ds(base + c * _CH, _CH)],
                     osems[slot])

  for c in (_NCH - 2, _NCH - 1):
    slot = c % 2
    pltpu.make_async_copy(vouts[slot],
                          out_hbm.at[pl.ds(base + c * _CH, _CH)],
                          osems[slot]).wait()


@jax.jit
def _run(x_flat, tbl):
  mesh = plsc.VectorSubcoreMesh(core_axis_name="c", subcore_axis_name="s")
  fn = pl.kernel(
      _sc_bucketize,
      out_type=jax.ShapeDtypeStruct((_N,), jnp.int32),
      mesh=mesh,
      compiler_params=pltpu.CompilerParams(needs_layout_passes=False),
      scratch_types=[
          pltpu.VMEM((32,), jnp.float32),
          pltpu.VMEM((_CH,), jnp.float32),
          pltpu.VMEM((_CH,), jnp.float32),
          pltpu.VMEM((_CH,), jnp.int32),
          pltpu.VMEM((_CH,), jnp.int32),
          pltpu.SemaphoreType.DMA,
          pltpu.SemaphoreType.DMA,
          pltpu.SemaphoreType.DMA,
          pltpu.SemaphoreType.DMA,
      ],
  )
  return fn(x_flat, tbl)


def kernel(inputs):
  tbl = jnp.asarray(_BOUNDS, dtype=jnp.float32)
  out = _run(inputs.reshape(_N), tbl)
  return out.reshape(inputs.shape)


# speedup vs baseline: 9.1427x; 1.7250x over previous
import functools

import jax
import jax.numpy as jnp
from jax import lax
from jax.experimental import pallas as pl
from jax.experimental.pallas import tpu as pltpu
from jax.experimental.pallas import tpu_sc as plsc

_BOUNDS = [-3.1, -2.9, -2.7, -2.5, -2.3, -2.1, -1.9, -1.7, -1.5, -1.3, -1.1,
           -0.9, -0.7, -0.5, -0.3, -0.1, 0.1, 0.3, 0.5, 0.7, 0.9, 1.1, 1.3,
           1.5, 1.7, 1.9, 2.1, 2.3, 2.5, 2.7, 2.9, 3.1]

_N = 4096 * 4096
_NW = 32
_PER_W = _N // _NW
_CH = 16384
_NCH = _PER_W // _CH
_VECS = _CH // 16


def _bucketize_chunk(vin, vout, tbl):
  @plsc.parallel_loop(0, _VECS, unroll=8)
  def body(i):
    v = vin[pl.ds(i * 16, 16)]
    t = v * 5.0 + 16.0
    t = jnp.minimum(jnp.maximum(t, 0.0), 31.0)
    j = t.astype(jnp.int32)
    b = plsc.load_gather(tbl, [j])
    vout[pl.ds(i * 16, 16)] = j + (v >= b).astype(jnp.int32)


def _sc_bucketize(x_hbm, tbl_hbm, out_hbm, tbl_v, vin0, vin1, vout0, vout1,
                  isem0, isem1, osem0, osem1):
  wid = lax.axis_index("s") * 2 + lax.axis_index("c")
  base = wid * _PER_W

  pltpu.sync_copy(tbl_hbm, tbl_v)

  vins = (vin0, vin1)
  vouts = (vout0, vout1)
  isems = (isem0, isem1)
  osems = (osem0, osem1)

  def start_in(c):
    slot = c % 2
    pltpu.async_copy(x_hbm.at[pl.ds(base + c * _CH, _CH)], vins[slot],
                     isems[slot])

  start_in(0)
  for c in range(_NCH):
    slot = c % 2
    if c + 1 < _NCH:
      start_in(c + 1)
    pltpu.make_async_copy(x_hbm.at[pl.ds(base + c * _CH, _CH)], vins[slot],
                          isems[slot]).wait()
    if c >= 2:
      pltpu.make_async_copy(vouts[slot],
                            out_hbm.at[pl.ds(base + (c - 2) * _CH, _CH)],
                            osems[slot]).wait()
    _bucketize_chunk(vins[slot], vouts[slot], tbl_v)
    pltpu.async_copy(vouts[slot], out_hbm.at[pl.ds(base + c * _CH, _CH)],
                     osems[slot])

  for c in (_NCH - 2, _NCH - 1):
    slot = c % 2
    pltpu.make_async_copy(vouts[slot],
                          out_hbm.at[pl.ds(base + c * _CH, _CH)],
                          osems[slot]).wait()


@jax.jit
def _run(x_flat, tbl):
  mesh = plsc.VectorSubcoreMesh(core_axis_name="c", subcore_axis_name="s")
  fn = pl.kernel(
      _sc_bucketize,
      out_type=jax.ShapeDtypeStruct((_N,), jnp.int32),
      mesh=mesh,
      compiler_params=pltpu.CompilerParams(needs_layout_passes=False),
      scratch_types=[
          pltpu.VMEM((32,), jnp.float32),
          pltpu.VMEM((_CH,), jnp.float32),
          pltpu.VMEM((_CH,), jnp.float32),
          pltpu.VMEM((_CH,), jnp.int32),
          pltpu.VMEM((_CH,), jnp.int32),
          pltpu.SemaphoreType.DMA,
          pltpu.SemaphoreType.DMA,
          pltpu.SemaphoreType.DMA,
          pltpu.SemaphoreType.DMA,
      ],
  )
  return fn(x_flat, tbl)


def kernel(inputs):
  tbl = jnp.asarray(_BOUNDS, dtype=jnp.float32)
  out = _run(inputs.reshape(_N), tbl)
  return out.reshape(inputs.shape)


# speedup vs baseline: 18.0058x; 1.9694x over previous
import functools

import jax
import jax.numpy as jnp
from jax import lax
from jax.experimental import pallas as pl
from jax.experimental.pallas import tpu as pltpu
from jax.experimental.pallas import tpu_sc as plsc

_BOUNDS = [-3.1, -2.9, -2.7, -2.5, -2.3, -2.1, -1.9, -1.7, -1.5, -1.3, -1.1,
           -0.9, -0.7, -0.5, -0.3, -0.1, 0.1, 0.3, 0.5, 0.7, 0.9, 1.1, 1.3,
           1.5, 1.7, 1.9, 2.1, 2.3, 2.5, 2.7, 2.9, 3.1]

_ROWS = 4096
_COLS = 4096
_NW = 32
_ROWS_W = _ROWS // _NW
_CR = 8
_NCH = _ROWS_W // _CR
_VECS = _COLS // 16


def _bucketize_chunk(vin, vout, tbl):
  @plsc.parallel_loop(0, _CR * _VECS, unroll=8)
  def body(i):
    r = i >> 8
    col = (i & (_VECS - 1)) * 16
    v = vin[r, pl.ds(col, 16)]
    t = v * 5.0 + 16.0
    t = jnp.minimum(jnp.maximum(t, 0.0), 31.0)
    j = t.astype(jnp.int32)
    b = plsc.load_gather(tbl, [j])
    vout[r, pl.ds(col, 16)] = j + (v >= b).astype(jnp.int32)


def _sc_bucketize(x_hbm, tbl_hbm, out_hbm, tbl_v, vin0, vin1, vout0,
                  isem0, isem1, osem0):
  wid = lax.axis_index("s") * 2 + lax.axis_index("c")
  base = wid * _ROWS_W

  pltpu.sync_copy(tbl_hbm, tbl_v)

  vins = (vin0, vin1)
  isems = (isem0, isem1)

  def start_in(c):
    slot = c % 2
    pltpu.async_copy(x_hbm.at[pl.ds(base + c * _CR, _CR)], vins[slot],
                     isems[slot])

  start_in(0)
  for c in range(_NCH):
    slot = c % 2
    if c + 1 < _NCH:
      start_in(c + 1)
    pltpu.make_async_copy(x_hbm.at[pl.ds(base + c * _CR, _CR)], vins[slot],
                          isems[slot]).wait()
    if c >= 1:
      pltpu.make_async_copy(vout0,
                            out_hbm.at[pl.ds(base + (c - 1) * _CR, _CR)],
                            osem0).wait()
    _bucketize_chunk(vins[slot], vout0, tbl_v)
    pltpu.async_copy(vout0, out_hbm.at[pl.ds(base + c * _CR, _CR)], osem0)

  pltpu.make_async_copy(vout0,
                        out_hbm.at[pl.ds(base + (_NCH - 1) * _CR, _CR)],
                        osem0).wait()


@jax.jit
def _run(x, tbl):
  mesh = plsc.VectorSubcoreMesh(core_axis_name="c", subcore_axis_name="s")
  fn = pl.kernel(
      _sc_bucketize,
      out_type=jax.ShapeDtypeStruct((_ROWS, _COLS), jnp.int32),
      mesh=mesh,
      compiler_params=pltpu.CompilerParams(needs_layout_passes=False),
      scratch_types=[
          pltpu.VMEM((32,), jnp.float32),
          pltpu.VMEM((_CR, _COLS), jnp.float32),
          pltpu.VMEM((_CR, _COLS), jnp.float32),
          pltpu.VMEM((_CR, _COLS), jnp.int32),
          pltpu.SemaphoreType.DMA,
          pltpu.SemaphoreType.DMA,
          pltpu.SemaphoreType.DMA,
      ],
  )
  return fn(x, tbl)


def kernel(inputs):
  tbl = jnp.asarray(_BOUNDS, dtype=jnp.float32)
  return _run(inputs, tbl)
